# baseline (speedup 1.0000x reference)
import functools

import jax
import jax.numpy as jnp
from jax import lax
from jax.experimental import pallas as pl
from jax.experimental.pallas import tpu as pltpu
from jax.experimental.pallas import tpu_sc as plsc


def _sc_agg_kernel(N_pad, D, NC, NS, C, G, NG):
    RPT = N_pad // NS
    mesh = plsc.VectorSubcoreMesh(core_axis_name="c", subcore_axis_name="s")

    @functools.partial(
        pl.kernel,
        mesh=mesh,
        out_type=jax.ShapeDtypeStruct((NC, NS, RPT, D), jnp.float32),
        scratch_types=[
            pltpu.VMEM((2, G, C), jnp.int32),
            pltpu.VMEM((2, G, C), jnp.int32),
            pltpu.VMEM((2, C, D), jnp.float32),
            pltpu.VMEM_SHARED((N_pad, D), jnp.float32),
            pltpu.SemaphoreType.DMA,
        ],
    )
    def sc_kernel(x_hbm, src_hbm, dst_hbm, out_hbm, src_v, dst_v, rows_v,
                  agg_sh, sem):
        c = lax.axis_index("c")
        s = lax.axis_index("s")
        wid = c * NS + s

        def gather_start(p, k, b):
            pltpu.async_copy(x_hbm.at[src_v.at[p, k]], rows_v.at[b], sem)

        def gather_wait(p, k, b):
            pltpu.make_async_copy(x_hbm.at[src_v.at[p, k]], rows_v.at[b],
                                  sem).wait()

        def scatter_add(p, k, b):
            pltpu.sync_copy(rows_v.at[b], agg_sh.at[dst_v.at[p, k]], add=True)

        def zfill(i, _):
            rows_v[0, i // (D // 16), pl.ds((i % (D // 16)) * 16, 16)] = (
                jnp.zeros((16,), jnp.float32))
            return 0
        lax.fori_loop(0, C * (D // 16), zfill, 0)
        r0 = s * RPT
        for k in range(RPT // C):
            pltpu.sync_copy(rows_v.at[0], agg_sh.at[pl.ds(r0 + k * C, C)])
        rem = RPT % C
        if rem:
            pltpu.sync_copy(rows_v.at[0, pl.ds(0, rem)],
                            agg_sh.at[pl.ds(r0 + (RPT // C) * C, rem)])
        plsc.subcore_barrier()

        pltpu.sync_copy(src_hbm.at[wid, 0], src_v.at[0])
        pltpu.sync_copy(dst_hbm.at[wid, 0], dst_v.at[0])
        gather_start(0, 0, 0)
        for g in range(NG):
            p = g % 2

            def pair(i, _, p=p):
                k0 = 2 * i
                gather_start(p, k0 + 1, 1)
                gather_wait(p, k0, 0)
                scatter_add(p, k0, 0)
                gather_start(p, k0 + 2, 0)
                gather_wait(p, k0 + 1, 1)
                scatter_add(p, k0 + 1, 1)
                return 0
            lax.fori_loop(0, G // 2 - 1, pair, 0)
            gather_start(p, G - 1, 1)
            gather_wait(p, G - 2, 0)
            scatter_add(p, G - 2, 0)
            if g + 1 < NG:
                pltpu.sync_copy(src_hbm.at[wid, g + 1], src_v.at[1 - p])
                pltpu.sync_copy(dst_hbm.at[wid, g + 1], dst_v.at[1 - p])
                gather_start(1 - p, 0, 0)
            gather_wait(p, G - 1, 1)
            scatter_add(p, G - 1, 1)
        plsc.subcore_barrier()

        pltpu.sync_copy(agg_sh.at[pl.ds(r0, RPT)], out_hbm.at[c, s])

    return sc_kernel


def _tc_finish_body(x_ref, a0_ref, a1_ref, wl_ref, wn_ref, b_ref, o_ref):
    x = x_ref[...]
    agg = a0_ref[...] + a1_ref[...]
    h = (jnp.dot(x, wl_ref[...], preferred_element_type=jnp.float32)
         + jnp.dot(agg, wn_ref[...], preferred_element_type=jnp.float32)
         + b_ref[...])
    o_ref[...] = jnp.tanh(h + x)


def kernel(x, edge_index, W_loop, W_neigh, b):
    N, D = x.shape
    E = edge_index.shape[1]

    NC, NS = 2, 16
    NW = NC * NS
    per_tile = E // NW
    C = 128
    G = 20
    NG = 4
    PT = C * G * NG
    N_pad = (N + C) // C * C

    pad = PT - per_tile
    src = jnp.concatenate(
        [edge_index[0].reshape(NW, per_tile),
         jnp.zeros((NW, pad), jnp.int32)], axis=1).reshape(NW, NG, G, C)
    dst = jnp.concatenate(
        [edge_index[1].reshape(NW, per_tile),
         jnp.full((NW, pad), N, jnp.int32)], axis=1).reshape(NW, NG, G, C)

    agg2 = _sc_agg_kernel(N_pad, D, NC, NS, C, G, NG)(x, src, dst)
    a0 = agg2[0].reshape(N_pad, D)
    a1 = agg2[1].reshape(N_pad, D)

    RB = 2000
    out = pl.pallas_call(
        _tc_finish_body,
        grid=(N // RB,),
        in_specs=[
            pl.BlockSpec((RB, D), lambda i: (i, 0)),
            pl.BlockSpec((RB, D), lambda i: (i, 0)),
            pl.BlockSpec((RB, D), lambda i: (i, 0)),
            pl.BlockSpec((D, D), lambda i: (0, 0)),
            pl.BlockSpec((D, D), lambda i: (0, 0)),
            pl.BlockSpec((1, D), lambda i: (0, 0)),
        ],
        out_specs=pl.BlockSpec((RB, D), lambda i: (i, 0)),
        out_shape=jax.ShapeDtypeStruct((N, D), jnp.float32),
    )(x, a0, a1, W_loop, W_neigh, b.reshape(1, D))
    return out
